# Initial kernel scaffold; baseline (speedup 1.0000x reference)
import jax
import jax.numpy as jnp
from jax.experimental import pallas as pl


def kernel(s):
    raise NotImplementedError("write your pallas kernel here")


# speedup vs baseline: 2.5520x; 2.5520x over previous
import functools

import jax
import jax.numpy as jnp
from jax.experimental import pallas as pl
from jax.experimental.pallas import tpu as pltpu

K = 2
TAU = 0.01
MAX_ITER = 100
TOL = 1e-3

_SORT_NET = [
    (0, 1), (2, 3), (4, 5), (6, 7),
    (0, 2), (1, 3), (4, 6), (5, 7),
    (1, 2), (5, 6),
    (0, 4), (1, 5), (2, 6), (3, 7),
    (2, 4), (3, 5),
    (1, 2), (3, 4), (5, 6),
]


def _gate_kernel(st_ref, g_ref):
    st = st_ref[...]

    cols = [st[j] for j in range(8)]
    for (a, b) in _SORT_NET:
        lo = jnp.minimum(cols[a], cols[b])
        hi = jnp.maximum(cols[a], cols[b])
        cols[a], cols[b] = lo, hi
    theta0 = cols[8 - K - 1][None, :]

    n = st.shape[1]

    def body(carry):
        theta, i, done = carry
        sig = jax.nn.sigmoid((st - theta) / TAU)
        f = jnp.sum(sig, axis=0, keepdims=True) - K
        new_done = (jnp.sum(f) / n) < TOL
        df = -(1.0 / TAU) * jnp.sum(sig * (1.0 - sig), axis=0, keepdims=True)
        theta_new = theta - f / df
        theta_out = jnp.where(new_done, theta, theta_new)
        return (theta_out, i + 1, new_done)

    def cond(carry):
        _, i, done = carry
        return jnp.logical_and(i < MAX_ITER, jnp.logical_not(done))

    theta, _, _ = jax.lax.while_loop(
        cond, body, (theta0, jnp.int32(0), jnp.bool_(False))
    )

    g_ref[...] = jax.nn.sigmoid((st - theta) / TAU)


@jax.jit
def kernel(s):
    st = s.T
    g_t = pl.pallas_call(
        _gate_kernel,
        out_shape=jax.ShapeDtypeStruct(st.shape, st.dtype),
        in_specs=[pl.BlockSpec(memory_space=pltpu.VMEM)],
        out_specs=pl.BlockSpec(memory_space=pltpu.VMEM),
    )(st)
    return g_t.T
